# baseline (speedup 1.0000x reference)
import jax
import jax.numpy as jnp
from jax.experimental import pallas as pl
from jax.experimental.pallas import tpu as pltpu

_P = 4194304
_CHUNK = 1 << 21
_NCHUNK = _P // _CHUNK
_TOTAL = 2 * _NCHUNK


def _body(a_smem, b_any, out_any, zbuf, inb, outb, zsem, in_sems, out_sems):
    nz = [a_smem[0] != 0.0, a_smem[1] != 0.0]

    def in_copy(c, slot):
        b, j = divmod(c, _NCHUNK)
        return pltpu.make_async_copy(
            b_any.at[b, pl.ds(0, 1), pl.ds(j * _CHUNK, _CHUNK)],
            inb.at[slot],
            in_sems.at[slot],
        )

    def out_copy(c, slot):
        b, j = divmod(c, _NCHUNK)
        return pltpu.make_async_copy(
            outb.at[slot],
            out_any.at[b, pl.ds(0, 1), pl.ds(j * _CHUNK, _CHUNK)],
            out_sems.at[slot],
        )

    def zero_copy(c):
        b, j = divmod(c, _NCHUNK)
        return pltpu.make_async_copy(
            zbuf,
            out_any.at[b, pl.ds(0, 1), pl.ds(j * _CHUNK, _CHUNK)],
            zsem,
        )

    for c in range(min(2, _TOTAL)):
        @pl.when(nz[c // _NCHUNK])
        def _():
            in_copy(c, c % 2).start()

    zbuf[...] = jnp.zeros((1, _CHUNK), jnp.float32)
    for c in range(_TOTAL):
        @pl.when(jnp.logical_not(nz[c // _NCHUNK]))
        def _():
            zero_copy(c).start()

    for c in range(_TOTAL):
        b = c // _NCHUNK
        s = c % 2

        @pl.when(nz[b])
        def _():
            in_copy(c, s).wait()
            if c >= 2:
                @pl.when(nz[(c - 2) // _NCHUNK])
                def _():
                    out_copy(c - 2, s).wait()
            outb[s] = a_smem[b] * inb[s]
            out_copy(c, s).start()

        if c + 2 < _TOTAL:
            @pl.when(nz[(c + 2) // _NCHUNK])
            def _():
                in_copy(c + 2, s).start()

    for c in range(_TOTAL):
        b = c // _NCHUNK
        if c + 2 < _TOTAL:
            pending = jnp.logical_and(
                nz[b], jnp.logical_not(nz[(c + 2) // _NCHUNK]))
        else:
            pending = nz[b]

        @pl.when(pending)
        def _():
            out_copy(c, c % 2).wait()

    for c in range(_TOTAL):
        @pl.when(jnp.logical_not(nz[c // _NCHUNK]))
        def _():
            zero_copy(c).wait()


def kernel(B, A):
    a2 = A.reshape(2)
    out = pl.pallas_call(
        _body,
        in_specs=[
            pl.BlockSpec(memory_space=pltpu.SMEM),
            pl.BlockSpec(memory_space=pl.ANY),
        ],
        out_specs=pl.BlockSpec(memory_space=pl.ANY),
        out_shape=jax.ShapeDtypeStruct((2, 1, _P), jnp.float32),
        scratch_shapes=[
            pltpu.VMEM((1, _CHUNK), jnp.float32),
            pltpu.VMEM((2, 1, _CHUNK), jnp.float32),
            pltpu.VMEM((2, 1, _CHUNK), jnp.float32),
            pltpu.SemaphoreType.DMA,
            pltpu.SemaphoreType.DMA((2,)),
            pltpu.SemaphoreType.DMA((2,)),
        ],
    )(a2, B)
    return out


# speedup vs baseline: 1.0739x; 1.0739x over previous
import jax
import jax.numpy as jnp
from jax.experimental import pallas as pl
from jax.experimental.pallas import tpu as pltpu

_P = 4194304
_CHUNK = 1 << 21
_NCHUNK = _P // _CHUNK
_TOTAL = 2 * _NCHUNK
_NBUF = 4
_LOOK = _NBUF - 1


def _body(a_smem, b_any, out_vmem, inb, sems):
    bi = pl.program_id(0)
    j = pl.program_id(1)
    i = bi * _NCHUNK + j

    def in_copy(b_idx, j_idx, slot):
        return pltpu.make_async_copy(
            b_any.at[b_idx, pl.ds(0, 1), pl.ds(j_idx * _CHUNK, _CHUNK)],
            inb.at[slot],
            sems.at[slot],
        )

    @pl.when(i == 0)
    def _():
        for c in range(min(_LOOK, _TOTAL)):
            cb, cj = divmod(c, _NCHUNK)

            @pl.when(a_smem[cb] != 0.0)
            def _():
                in_copy(cb, cj, c % _NBUF).start()

    i2 = i + _LOOK
    b2 = jnp.minimum(i2 // _NCHUNK, 1)
    j2 = i2 % _NCHUNK

    @pl.when(jnp.logical_and(i2 < _TOTAL, a_smem[b2] != 0.0))
    def _():
        in_copy(b2, j2, i2 % _NBUF).start()

    a = a_smem[bi]

    @pl.when(a != 0.0)
    def _():
        in_copy(bi, j, i % _NBUF).wait()
        out_vmem[0] = a * inb[i % _NBUF]

    @pl.when(a == 0.0)
    def _():
        out_vmem[0] = jnp.zeros((1, _CHUNK), jnp.float32)


def kernel(B, A):
    a2 = A.reshape(2)
    out = pl.pallas_call(
        _body,
        grid=(2, _NCHUNK),
        in_specs=[
            pl.BlockSpec(memory_space=pltpu.SMEM),
            pl.BlockSpec(memory_space=pl.ANY),
        ],
        out_specs=pl.BlockSpec((1, 1, _CHUNK), lambda b, j: (b, 0, j)),
        out_shape=jax.ShapeDtypeStruct((2, 1, _P), jnp.float32),
        scratch_shapes=[
            pltpu.VMEM((_NBUF, 1, _CHUNK), jnp.float32),
            pltpu.SemaphoreType.DMA((_NBUF,)),
        ],
    )(a2, B)
    return out
